# baseline (speedup 1.0000x reference)
import jax
import jax.numpy as jnp
from jax import lax
from jax.experimental import pallas as pl
from jax.experimental.pallas import tpu as pltpu
from jax.experimental.pallas import tpu_sc as plsc

EPS_W = 1e-06

B, C, H, W = 16, 3, 512, 512
HW = H * W
NC, NS, L = 2, 16, 16
PX_TILE = HW // NS
CHUNK = 2048
NCHUNK = PX_TILE // CHUNK
IMGS_PER_CORE = B // NC


def _body(im_hbm, flow_hbm, out_hbm,
          accR, accG, accB, accW,
          flow_v, imr, img_, imb, wbuf,
          idx0, idx1, idx2, idx3,
          val0, val1, val2, val3, zbuf, dsem):
    cid = lax.axis_index("c")
    sid = lax.axis_index("s")
    tile_base = sid * PX_TILE

    iota = lax.iota(jnp.int32, L)
    iota2 = iota * 2

    idx_refs = (idx0, idx1, idx2, idx3)
    val_refs = (val0, val1, val2, val3)
    im_refs = (imr, img_, imb)
    acc_refs = (accR, accG, accB, accW)

    zvec = jnp.zeros((L,), jnp.float32)

    def zero_vec(j, _):
        zbuf[pl.ds(j * L, L)] = zvec
        return 0

    lax.fori_loop(0, CHUNK // L, zero_vec, 0)

    def one_image(i, _):
        img = 2 * i + cid

        hs = []
        for k in range(NCHUNK):
            off = tile_base + k * CHUNK
            for a in range(4):
                hs.append(pltpu.async_copy(
                    zbuf, acc_refs[a].at[pl.ds(off, CHUNK)], dsem))
        for h in hs:
            h.wait()
        plsc.subcore_barrier()

        def splat_chunk(k, _):
            px0 = tile_base + k * CHUNK
            hs = [pltpu.async_copy(
                flow_hbm.at[pl.ds((img * HW + px0) * 2, 2 * CHUNK)],
                flow_v, dsem)]
            for ch in range(C):
                hs.append(pltpu.async_copy(
                    im_hbm.at[pl.ds((img * C + ch) * HW + px0, CHUNK)],
                    im_refs[ch], dsem))
            for h in hs:
                h.wait()

            def splat_vec(j, _):
                off = j * L
                rowv = off + iota
                fbase = 2 * off + iota2
                dx = plsc.load_gather(flow_v, [fbase])
                dy = plsc.load_gather(flow_v, [fbase + 1])
                p = px0 + rowv
                xi = lax.bitwise_and(p, W - 1)
                yi = lax.shift_right_logical(p, 9)
                tx = xi.astype(jnp.float32) + dx
                ty = yi.astype(jnp.float32) + dy
                ti = tx.astype(jnp.int32)
                tf = ti.astype(jnp.float32)
                fx = jnp.where(tf > tx, tf - 1.0, tf)
                x0 = fx.astype(jnp.int32)
                ti = ty.astype(jnp.int32)
                tf = ti.astype(jnp.float32)
                fy = jnp.where(tf > ty, tf - 1.0, tf)
                y0 = fy.astype(jnp.int32)
                wx1 = tx - fx
                wx0 = 1.0 - wx1
                wy1 = ty - fy
                wy0 = 1.0 - wy1
                ax0 = jnp.where((x0 >= 0) & (x0 < W), wx0, 0.0)
                ax1 = jnp.where((x0 >= -1) & (x0 < W - 1), wx1, 0.0)
                ay0 = jnp.where((y0 >= 0) & (y0 < H), wy0, 0.0)
                ay1 = jnp.where((y0 >= -1) & (y0 < H - 1), wy1, 0.0)
                x0c = jnp.maximum(jnp.minimum(x0, W - 1), 0)
                x1c = jnp.maximum(jnp.minimum(x0 + 1, W - 1), 0)
                y0c = jnp.maximum(jnp.minimum(y0, H - 1), 0)
                y1c = jnp.maximum(jnp.minimum(y0 + 1, H - 1), 0)
                ys0 = lax.shift_left(y0c, 9)
                ys1 = lax.shift_left(y1c, 9)
                r = imr[pl.ds(off, L)]
                g = img_[pl.ds(off, L)]
                b = imb[pl.ds(off, L)]
                corners = ((ax0, ay0, x0c, ys0), (ax1, ay0, x1c, ys0),
                           (ax0, ay1, x0c, ys1), (ax1, ay1, x1c, ys1))
                for cnum in range(4):
                    ax, ay, xc, ys = corners[cnum]
                    w = ax * ay
                    idx_refs[cnum][pl.ds(off, L)] = ys + xc
                    vref = val_refs[cnum]
                    vref[0, pl.ds(off, L)] = w * r
                    vref[1, pl.ds(off, L)] = w * g
                    vref[2, pl.ds(off, L)] = w * b
                    vref[3, pl.ds(off, L)] = w
                return 0

            lax.fori_loop(0, CHUNK // L, splat_vec, 0)
            hs = []
            for cnum in range(4):
                for a in range(4):
                    hs.append(pltpu.async_copy(
                        val_refs[cnum].at[a],
                        acc_refs[a].at[idx_refs[cnum]], dsem, add=True))
            for h in hs:
                h.wait()
            return 0

        lax.fori_loop(0, NCHUNK, splat_chunk, 0)
        plsc.subcore_barrier()

        def rescale_chunk(k, _):
            px0 = tile_base + k * CHUNK
            hs = []
            for ch in range(C):
                hs.append(pltpu.async_copy(
                    acc_refs[ch].at[pl.ds(px0, CHUNK)], im_refs[ch], dsem))
            hs.append(pltpu.async_copy(accW.at[pl.ds(px0, CHUNK)], wbuf,
                                       dsem))
            for h in hs:
                h.wait()

            def rescale_vec(j, _):
                off = j * L
                aw = wbuf[pl.ds(off, L)]
                inv = 1.0 / jnp.where(aw < EPS_W, 1.0, aw)
                imr[pl.ds(off, L)] = imr[pl.ds(off, L)] * inv
                img_[pl.ds(off, L)] = img_[pl.ds(off, L)] * inv
                imb[pl.ds(off, L)] = imb[pl.ds(off, L)] * inv
                return 0

            lax.fori_loop(0, CHUNK // L, rescale_vec, 0)
            hs = []
            for ch in range(C):
                hs.append(pltpu.async_copy(
                    im_refs[ch],
                    out_hbm.at[pl.ds((img * C + ch) * HW + px0, CHUNK)],
                    dsem))
            for h in hs:
                h.wait()
            return 0

        lax.fori_loop(0, NCHUNK, rescale_chunk, 0)
        plsc.subcore_barrier()
        return 0

    lax.fori_loop(0, IMGS_PER_CORE, one_image, 0)


@jax.jit
def kernel(im0, flow):
    im_flat = im0.reshape(B * C * HW)
    flow_flat = flow.reshape(B * HW * 2)

    mesh = plsc.VectorSubcoreMesh(core_axis_name="c", subcore_axis_name="s",
                                  num_cores=NC, num_subcores=NS)
    warp = pl.kernel(
        _body,
        out_type=jax.ShapeDtypeStruct((B * C * HW,), jnp.float32),
        mesh=mesh,
        compiler_params=pltpu.CompilerParams(needs_layout_passes=False,
                                             use_tc_tiling_on_sc=False),
        scratch_types=[
            pltpu.VMEM_SHARED((HW,), jnp.float32),
            pltpu.VMEM_SHARED((HW,), jnp.float32),
            pltpu.VMEM_SHARED((HW,), jnp.float32),
            pltpu.VMEM_SHARED((HW,), jnp.float32),
            pltpu.VMEM((2 * CHUNK,), jnp.float32),
            pltpu.VMEM((CHUNK,), jnp.float32),
            pltpu.VMEM((CHUNK,), jnp.float32),
            pltpu.VMEM((CHUNK,), jnp.float32),
            pltpu.VMEM((CHUNK,), jnp.float32),
            pltpu.VMEM((CHUNK,), jnp.int32),
            pltpu.VMEM((CHUNK,), jnp.int32),
            pltpu.VMEM((CHUNK,), jnp.int32),
            pltpu.VMEM((CHUNK,), jnp.int32),
            pltpu.VMEM((4, CHUNK), jnp.float32),
            pltpu.VMEM((4, CHUNK), jnp.float32),
            pltpu.VMEM((4, CHUNK), jnp.float32),
            pltpu.VMEM((4, CHUNK), jnp.float32),
            pltpu.VMEM((CHUNK,), jnp.float32),
            pltpu.SemaphoreType.DMA,
        ],
    )
    out = warp(im_flat, flow_flat)
    return out.reshape(B, C, H, W)


# speedup vs baseline: 4.9123x; 4.9123x over previous
import jax
import jax.numpy as jnp
from jax import lax
from jax.experimental import pallas as pl
from jax.experimental.pallas import tpu as pltpu
from jax.experimental.pallas import tpu_sc as plsc

EPS_W = 1e-06

B, C, H, W = 16, 3, 512, 512
HW = H * W
NC, NS, L = 2, 16, 16
PX_TILE = HW // NS
CHUNK = 2048
NCHUNK = PX_TILE // CHUNK
IMGS_PER_CORE = B // NC


def _body(im_hbm, flow_hbm, out_hbm,
          accR, accG, accB, accW,
          dxv, dyv, imr, img_, imb, wbuf,
          idx0, idx1, idx2, idx3,
          val0, val1, val2, val3, zbuf, dsem):
    cid = lax.axis_index("c")
    sid = lax.axis_index("s")
    tile_base = sid * PX_TILE

    iota = lax.iota(jnp.int32, L)

    idx_refs = (idx0, idx1, idx2, idx3)
    val_refs = (val0, val1, val2, val3)
    im_refs = (imr, img_, imb)
    acc_refs = (accR, accG, accB, accW)

    zvec = jnp.zeros((L,), jnp.float32)

    def zero_vec(j, _):
        zbuf[pl.ds(j * L, L)] = zvec
        return 0

    lax.fori_loop(0, CHUNK // L, zero_vec, 0)

    def one_image(i, _):
        img = 2 * i + cid

        def zero_chunk(k, _):
            off = tile_base + k * CHUNK
            for a in range(4):
                pltpu.sync_copy(zbuf, acc_refs[a].at[pl.ds(off, CHUNK)])
            return 0

        lax.fori_loop(0, NCHUNK, zero_chunk, 0)
        plsc.subcore_barrier()

        def splat_chunk(k, _):
            px0 = tile_base + k * CHUNK
            pltpu.sync_copy(flow_hbm.at[pl.ds((img * 2) * HW + px0, CHUNK)],
                            dxv)
            pltpu.sync_copy(flow_hbm.at[pl.ds((img * 2 + 1) * HW + px0,
                                              CHUNK)], dyv)
            for ch in range(C):
                pltpu.sync_copy(
                    im_hbm.at[pl.ds((img * C + ch) * HW + px0, CHUNK)],
                    im_refs[ch])

            def splat_vec(j, _):
                off = j * L
                rowv = off + iota
                dx = dxv[pl.ds(off, L)]
                dy = dyv[pl.ds(off, L)]
                p = px0 + rowv
                xi = lax.bitwise_and(p, W - 1)
                yi = lax.shift_right_logical(p, 9)
                tx = xi.astype(jnp.float32) + dx
                ty = yi.astype(jnp.float32) + dy
                ti = tx.astype(jnp.int32)
                tf = ti.astype(jnp.float32)
                fx = jnp.where(tf > tx, tf - 1.0, tf)
                x0 = fx.astype(jnp.int32)
                ti = ty.astype(jnp.int32)
                tf = ti.astype(jnp.float32)
                fy = jnp.where(tf > ty, tf - 1.0, tf)
                y0 = fy.astype(jnp.int32)
                wx1 = tx - fx
                wx0 = 1.0 - wx1
                wy1 = ty - fy
                wy0 = 1.0 - wy1
                ax0 = jnp.where((x0 >= 0) & (x0 < W), wx0, 0.0)
                ax1 = jnp.where((x0 >= -1) & (x0 < W - 1), wx1, 0.0)
                ay0 = jnp.where((y0 >= 0) & (y0 < H), wy0, 0.0)
                ay1 = jnp.where((y0 >= -1) & (y0 < H - 1), wy1, 0.0)
                x0c = jnp.maximum(jnp.minimum(x0, W - 1), 0)
                x1c = jnp.maximum(jnp.minimum(x0 + 1, W - 1), 0)
                y0c = jnp.maximum(jnp.minimum(y0, H - 1), 0)
                y1c = jnp.maximum(jnp.minimum(y0 + 1, H - 1), 0)
                ys0 = lax.shift_left(y0c, 9)
                ys1 = lax.shift_left(y1c, 9)
                r = imr[pl.ds(off, L)]
                g = img_[pl.ds(off, L)]
                b = imb[pl.ds(off, L)]
                corners = ((ax0, ay0, x0c, ys0), (ax1, ay0, x1c, ys0),
                           (ax0, ay1, x0c, ys1), (ax1, ay1, x1c, ys1))
                for cnum in range(4):
                    ax, ay, xc, ys = corners[cnum]
                    w = ax * ay
                    idx_refs[cnum][pl.ds(off, L)] = ys + xc
                    vref = val_refs[cnum]
                    vref[0, pl.ds(off, L)] = w * r
                    vref[1, pl.ds(off, L)] = w * g
                    vref[2, pl.ds(off, L)] = w * b
                    vref[3, pl.ds(off, L)] = w
                return 0

            lax.fori_loop(0, CHUNK // L, splat_vec, 0)
            hs = []
            for cnum in range(4):
                for a in range(4):
                    hs.append(pltpu.async_copy(
                        val_refs[cnum].at[a],
                        acc_refs[a].at[idx_refs[cnum]], dsem, add=True))
            for h in hs:
                h.wait()
            return 0

        lax.fori_loop(0, NCHUNK, splat_chunk, 0)
        plsc.subcore_barrier()

        def rescale_chunk(k, _):
            px0 = tile_base + k * CHUNK
            for ch in range(C):
                pltpu.sync_copy(acc_refs[ch].at[pl.ds(px0, CHUNK)],
                                im_refs[ch])
            pltpu.sync_copy(accW.at[pl.ds(px0, CHUNK)], wbuf)

            def rescale_vec(j, _):
                off = j * L
                aw = wbuf[pl.ds(off, L)]
                inv = 1.0 / jnp.where(aw < EPS_W, 1.0, aw)
                imr[pl.ds(off, L)] = imr[pl.ds(off, L)] * inv
                img_[pl.ds(off, L)] = img_[pl.ds(off, L)] * inv
                imb[pl.ds(off, L)] = imb[pl.ds(off, L)] * inv
                return 0

            lax.fori_loop(0, CHUNK // L, rescale_vec, 0)
            for ch in range(C):
                pltpu.sync_copy(
                    im_refs[ch],
                    out_hbm.at[pl.ds((img * C + ch) * HW + px0, CHUNK)])
            return 0

        lax.fori_loop(0, NCHUNK, rescale_chunk, 0)
        plsc.subcore_barrier()
        return 0

    lax.fori_loop(0, IMGS_PER_CORE, one_image, 0)


@jax.jit
def kernel(im0, flow):
    im_flat = im0.reshape(B * C * HW)
    flow_flat = jnp.transpose(flow.reshape(B, HW, 2),
                              (0, 2, 1)).reshape(B * 2 * HW)

    mesh = plsc.VectorSubcoreMesh(core_axis_name="c", subcore_axis_name="s",
                                  num_cores=NC, num_subcores=NS)
    warp = pl.kernel(
        _body,
        out_type=jax.ShapeDtypeStruct((B * C * HW,), jnp.float32),
        mesh=mesh,
        compiler_params=pltpu.CompilerParams(needs_layout_passes=False, use_tc_tiling_on_sc=False),
        scratch_types=[
            pltpu.VMEM_SHARED((HW,), jnp.float32),
            pltpu.VMEM_SHARED((HW,), jnp.float32),
            pltpu.VMEM_SHARED((HW,), jnp.float32),
            pltpu.VMEM_SHARED((HW,), jnp.float32),
            pltpu.VMEM((CHUNK,), jnp.float32),
            pltpu.VMEM((CHUNK,), jnp.float32),
            pltpu.VMEM((CHUNK,), jnp.float32),
            pltpu.VMEM((CHUNK,), jnp.float32),
            pltpu.VMEM((CHUNK,), jnp.float32),
            pltpu.VMEM((CHUNK,), jnp.float32),
            pltpu.VMEM((CHUNK,), jnp.int32),
            pltpu.VMEM((CHUNK,), jnp.int32),
            pltpu.VMEM((CHUNK,), jnp.int32),
            pltpu.VMEM((CHUNK,), jnp.int32),
            pltpu.VMEM((4, CHUNK), jnp.float32),
            pltpu.VMEM((4, CHUNK), jnp.float32),
            pltpu.VMEM((4, CHUNK), jnp.float32),
            pltpu.VMEM((4, CHUNK), jnp.float32),
            pltpu.VMEM((CHUNK,), jnp.float32),
            pltpu.SemaphoreType.DMA,
        ],
    )
    out = warp(im_flat, flow_flat)
    return out.reshape(B, C, H, W)


# speedup vs baseline: 6.0410x; 1.2298x over previous
import jax
import jax.numpy as jnp
from jax import lax
from jax.experimental import pallas as pl
from jax.experimental.pallas import tpu as pltpu
from jax.experimental.pallas import tpu_sc as plsc

EPS_W = 1e-06

B, C, H, W = 16, 3, 512, 512
HW = H * W
NC, NS, L = 2, 16, 16
PX_TILE = HW // NS
CHUNK = 1024
NCHUNK = PX_TILE // CHUNK
IMGS_PER_CORE = B // NC


def _body(im_hbm, flow_hbm, out_hbm,
          accR, accG, accB, accW,
          dxv0, dyv0, imr0, img0, imb0,
          dxv1, dyv1, imr1, img1, imb1,
          idxA0, idxA1, idxA2, idxA3,
          idxB0, idxB1, idxB2, idxB3,
          valA0, valA1, valA2, valA3,
          valB0, valB1, valB2, valB3,
          wbuf, zbuf, sem_in, sem_sc0, sem_sc1):
    cid = lax.axis_index("c")
    sid = lax.axis_index("s")
    tile_base = sid * PX_TILE

    iota = lax.iota(jnp.int32, L)

    in_sets = ((dxv0, dyv0, imr0, img0, imb0),
               (dxv1, dyv1, imr1, img1, imb1))
    idx_sets = ((idxA0, idxA1, idxA2, idxA3), (idxB0, idxB1, idxB2, idxB3))
    val_sets = ((valA0, valA1, valA2, valA3), (valB0, valB1, valB2, valB3))
    sc_sems = (sem_sc0, sem_sc1)
    acc_refs = (accR, accG, accB, accW)

    zvec = jnp.zeros((L,), jnp.float32)

    def zero_vec(j, _):
        zbuf[pl.ds(j * L, L)] = zvec
        return 0

    lax.fori_loop(0, CHUNK // L, zero_vec, 0)

    def fire_inputs(img, k):
        px0 = tile_base + k * CHUNK
        dxv, dyv, imr, img_, imb = in_sets[k % 2]
        hs = [pltpu.async_copy(
            flow_hbm.at[pl.ds((img * 2) * HW + px0, CHUNK)], dxv, sem_in),
            pltpu.async_copy(
            flow_hbm.at[pl.ds((img * 2 + 1) * HW + px0, CHUNK)], dyv,
            sem_in)]
        for ch, dst in enumerate((imr, img_, imb)):
            hs.append(pltpu.async_copy(
                im_hbm.at[pl.ds((img * C + ch) * HW + px0, CHUNK)], dst,
                sem_in))
        return hs

    def compute_chunk(k):
        px0 = tile_base + k * CHUNK
        dxv, dyv, imr, img_, imb = in_sets[k % 2]
        idx_refs = idx_sets[k % 2]
        val_refs = val_sets[k % 2]

        def splat_vec(j, _):
            off = j * L
            rowv = off + iota
            dx = dxv[pl.ds(off, L)]
            dy = dyv[pl.ds(off, L)]
            p = px0 + rowv
            xi = lax.bitwise_and(p, W - 1)
            yi = lax.shift_right_logical(p, 9)
            tx = xi.astype(jnp.float32) + dx
            ty = yi.astype(jnp.float32) + dy
            ti = tx.astype(jnp.int32)
            tf = ti.astype(jnp.float32)
            fx = jnp.where(tf > tx, tf - 1.0, tf)
            x0 = fx.astype(jnp.int32)
            ti = ty.astype(jnp.int32)
            tf = ti.astype(jnp.float32)
            fy = jnp.where(tf > ty, tf - 1.0, tf)
            y0 = fy.astype(jnp.int32)
            wx1 = tx - fx
            wx0 = 1.0 - wx1
            wy1 = ty - fy
            wy0 = 1.0 - wy1
            ax0 = jnp.where((x0 >= 0) & (x0 < W), wx0, 0.0)
            ax1 = jnp.where((x0 >= -1) & (x0 < W - 1), wx1, 0.0)
            ay0 = jnp.where((y0 >= 0) & (y0 < H), wy0, 0.0)
            ay1 = jnp.where((y0 >= -1) & (y0 < H - 1), wy1, 0.0)
            x0c = jnp.maximum(jnp.minimum(x0, W - 1), 0)
            x1c = jnp.maximum(jnp.minimum(x0 + 1, W - 1), 0)
            y0c = jnp.maximum(jnp.minimum(y0, H - 1), 0)
            y1c = jnp.maximum(jnp.minimum(y0 + 1, H - 1), 0)
            ys0 = lax.shift_left(y0c, 9)
            ys1 = lax.shift_left(y1c, 9)
            r = imr[pl.ds(off, L)]
            g = img_[pl.ds(off, L)]
            b = imb[pl.ds(off, L)]
            corners = ((ax0, ay0, x0c, ys0), (ax1, ay0, x1c, ys0),
                       (ax0, ay1, x0c, ys1), (ax1, ay1, x1c, ys1))
            for cnum in range(4):
                ax, ay, xc, ys = corners[cnum]
                w = ax * ay
                idx_refs[cnum][pl.ds(off, L)] = ys + xc
                vref = val_refs[cnum]
                vref[0, pl.ds(off, L)] = w * r
                vref[1, pl.ds(off, L)] = w * g
                vref[2, pl.ds(off, L)] = w * b
                vref[3, pl.ds(off, L)] = w
            return 0

        lax.fori_loop(0, CHUNK // L, splat_vec, 0)

    def fire_scatters(k):
        idx_refs = idx_sets[k % 2]
        val_refs = val_sets[k % 2]
        sem = sc_sems[k % 2]
        hs = []
        for cnum in range(4):
            for a in range(4):
                hs.append(pltpu.async_copy(
                    val_refs[cnum].at[a],
                    acc_refs[a].at[idx_refs[cnum]], sem, add=True))
        return hs

    def one_image(i, _):
        img = 2 * i + cid

        def zero_chunk(k, _):
            off = tile_base + k * CHUNK
            for a in range(4):
                pltpu.sync_copy(zbuf, acc_refs[a].at[pl.ds(off, CHUNK)])
            return 0

        lax.fori_loop(0, NCHUNK, zero_chunk, 0)
        plsc.subcore_barrier()

        in_hs = fire_inputs(img, 0)
        sc_hs = {}
        for k in range(NCHUNK):
            for h in in_hs:
                h.wait()
            if k + 1 < NCHUNK:
                in_hs = fire_inputs(img, k + 1)
            if k - 2 in sc_hs:
                for h in sc_hs.pop(k - 2):
                    h.wait()
            compute_chunk(k)
            sc_hs[k] = fire_scatters(k)
        for hs in sc_hs.values():
            for h in hs:
                h.wait()
        plsc.subcore_barrier()

        def rescale_chunk(k, _):
            px0 = tile_base + k * CHUNK
            imr, img_, imb = imr0, img0, imb0
            for ch, dst in enumerate((imr, img_, imb)):
                pltpu.sync_copy(acc_refs[ch].at[pl.ds(px0, CHUNK)], dst)
            pltpu.sync_copy(accW.at[pl.ds(px0, CHUNK)], wbuf)

            def rescale_vec(j, _):
                off = j * L
                aw = wbuf[pl.ds(off, L)]
                inv = 1.0 / jnp.where(aw < EPS_W, 1.0, aw)
                imr[pl.ds(off, L)] = imr[pl.ds(off, L)] * inv
                img_[pl.ds(off, L)] = img_[pl.ds(off, L)] * inv
                imb[pl.ds(off, L)] = imb[pl.ds(off, L)] * inv
                return 0

            lax.fori_loop(0, CHUNK // L, rescale_vec, 0)
            for ch, src in enumerate((imr, img_, imb)):
                pltpu.sync_copy(
                    src, out_hbm.at[pl.ds((img * C + ch) * HW + px0, CHUNK)])
            return 0

        lax.fori_loop(0, NCHUNK, rescale_chunk, 0)
        plsc.subcore_barrier()
        return 0

    lax.fori_loop(0, IMGS_PER_CORE, one_image, 0)


@jax.jit
def kernel(im0, flow):
    im_flat = im0.reshape(B * C * HW)
    flow_flat = jnp.transpose(flow.reshape(B, HW, 2),
                              (0, 2, 1)).reshape(B * 2 * HW)

    mesh = plsc.VectorSubcoreMesh(core_axis_name="c", subcore_axis_name="s",
                                  num_cores=NC, num_subcores=NS)
    f32 = jnp.float32
    i32 = jnp.int32
    warp = pl.kernel(
        _body,
        out_type=jax.ShapeDtypeStruct((B * C * HW,), f32),
        mesh=mesh,
        compiler_params=pltpu.CompilerParams(needs_layout_passes=False,
                                             use_tc_tiling_on_sc=False),
        scratch_types=(
            [pltpu.VMEM_SHARED((HW,), f32)] * 4
            + [pltpu.VMEM((CHUNK,), f32)] * 10
            + [pltpu.VMEM((CHUNK,), i32)] * 8
            + [pltpu.VMEM((4, CHUNK), f32)] * 8
            + [pltpu.VMEM((CHUNK,), f32)] * 2
            + [pltpu.SemaphoreType.DMA] * 3
        ),
    )
    out = warp(im_flat, flow_flat)
    return out.reshape(B, C, H, W)


# speedup vs baseline: 6.0985x; 1.0095x over previous
import jax
import jax.numpy as jnp
from jax import lax
from jax.experimental import pallas as pl
from jax.experimental.pallas import tpu as pltpu
from jax.experimental.pallas import tpu_sc as plsc

EPS_W = 1e-06

B, C, H, W = 16, 3, 512, 512
HW = H * W
NC, NS, L = 2, 16, 16
PX_TILE = HW // NS
CHUNK = 1024
NCHUNK = PX_TILE // CHUNK
IMGS_PER_CORE = B // NC
ROWS = CHUNK // W


def _body(im_hbm, flow_hbm, out_hbm,
          accR, accG, accB, accW,
          dxv0, dyv0, imr0, img0, imb0,
          dxv1, dyv1, imr1, img1, imb1,
          idxA0, idxA1, idxA2, idxA3,
          idxB0, idxB1, idxB2, idxB3,
          valA0, valA1, valA2, valA3,
          valB0, valB1, valB2, valB3,
          wbuf, zbuf, sem_in, sem_sc0, sem_sc1):
    cid = lax.axis_index("c")
    sid = lax.axis_index("s")
    tile_base = sid * PX_TILE

    iota = lax.iota(jnp.int32, L)

    in_sets = ((dxv0, dyv0, imr0, img0, imb0),
               (dxv1, dyv1, imr1, img1, imb1))
    idx_sets = ((idxA0, idxA1, idxA2, idxA3), (idxB0, idxB1, idxB2, idxB3))
    val_sets = ((valA0, valA1, valA2, valA3), (valB0, valB1, valB2, valB3))
    sc_sems = (sem_sc0, sem_sc1)
    acc_refs = (accR, accG, accB, accW)

    zvec = jnp.zeros((L,), jnp.float32)

    def zero_vec(j, _):
        zbuf[pl.ds(j * L, L)] = zvec
        return 0

    lax.fori_loop(0, CHUNK // L, zero_vec, 0)

    def fire_inputs(img, k):
        px0 = tile_base + k * CHUNK
        row0 = px0 // W
        dxv, dyv, imr, img_, imb = in_sets[k % 2]
        hs = [pltpu.async_copy(
            flow_hbm.at[pl.ds((img * 2) * HW + px0, CHUNK)], dxv, sem_in),
            pltpu.async_copy(
            flow_hbm.at[pl.ds((img * 2 + 1) * HW + px0, CHUNK)], dyv,
            sem_in)]
        for ch, dst in enumerate((imr, img_, imb)):
            hs.append(pltpu.async_copy(
                im_hbm.at[img, ch, pl.ds(row0, ROWS), :], dst, sem_in))
        return hs

    def compute_chunk(k):
        px0 = tile_base + k * CHUNK
        dxv, dyv, imr, img_, imb = in_sets[k % 2]
        idx_refs = idx_sets[k % 2]
        val_refs = val_sets[k % 2]

        def splat_vec(j, _):
            off = j * L
            rowv = off + iota
            dx = dxv[pl.ds(off, L)]
            dy = dyv[pl.ds(off, L)]
            p = px0 + rowv
            xi = lax.bitwise_and(p, W - 1)
            yi = lax.shift_right_logical(p, 9)
            tx = xi.astype(jnp.float32) + dx
            ty = yi.astype(jnp.float32) + dy
            ti = tx.astype(jnp.int32)
            tf = ti.astype(jnp.float32)
            fx = jnp.where(tf > tx, tf - 1.0, tf)
            x0 = fx.astype(jnp.int32)
            ti = ty.astype(jnp.int32)
            tf = ti.astype(jnp.float32)
            fy = jnp.where(tf > ty, tf - 1.0, tf)
            y0 = fy.astype(jnp.int32)
            wx1 = tx - fx
            wx0 = 1.0 - wx1
            wy1 = ty - fy
            wy0 = 1.0 - wy1
            ax0 = jnp.where((x0 >= 0) & (x0 < W), wx0, 0.0)
            ax1 = jnp.where((x0 >= -1) & (x0 < W - 1), wx1, 0.0)
            ay0 = jnp.where((y0 >= 0) & (y0 < H), wy0, 0.0)
            ay1 = jnp.where((y0 >= -1) & (y0 < H - 1), wy1, 0.0)
            x0c = jnp.maximum(jnp.minimum(x0, W - 1), 0)
            x1c = jnp.maximum(jnp.minimum(x0 + 1, W - 1), 0)
            y0c = jnp.maximum(jnp.minimum(y0, H - 1), 0)
            y1c = jnp.maximum(jnp.minimum(y0 + 1, H - 1), 0)
            ys0 = lax.shift_left(y0c, 9)
            ys1 = lax.shift_left(y1c, 9)
            r = imr[off // W, pl.ds(off % W, L)]
            g = img_[off // W, pl.ds(off % W, L)]
            b = imb[off // W, pl.ds(off % W, L)]
            corners = ((ax0, ay0, x0c, ys0), (ax1, ay0, x1c, ys0),
                       (ax0, ay1, x0c, ys1), (ax1, ay1, x1c, ys1))
            for cnum in range(4):
                ax, ay, xc, ys = corners[cnum]
                w = ax * ay
                idx_refs[cnum][pl.ds(off, L)] = ys + xc
                vref = val_refs[cnum]
                vref[0, pl.ds(off, L)] = w * r
                vref[1, pl.ds(off, L)] = w * g
                vref[2, pl.ds(off, L)] = w * b
                vref[3, pl.ds(off, L)] = w
            return 0

        lax.fori_loop(0, CHUNK // L, splat_vec, 0)

    def fire_scatters(k):
        idx_refs = idx_sets[k % 2]
        val_refs = val_sets[k % 2]
        sem = sc_sems[k % 2]
        hs = []
        for cnum in range(4):
            for a in range(4):
                hs.append(pltpu.async_copy(
                    val_refs[cnum].at[a],
                    acc_refs[a].at[idx_refs[cnum]], sem, add=True))
        return hs

    def one_image(i, _):
        img = 2 * i + cid

        def zero_chunk(k, _):
            off = tile_base + k * CHUNK
            for a in range(4):
                pltpu.sync_copy(zbuf, acc_refs[a].at[pl.ds(off, CHUNK)])
            return 0

        lax.fori_loop(0, NCHUNK, zero_chunk, 0)
        plsc.subcore_barrier()

        in_hs = fire_inputs(img, 0)
        sc_hs = {}
        for k in range(NCHUNK):
            for h in in_hs:
                h.wait()
            if k + 1 < NCHUNK:
                in_hs = fire_inputs(img, k + 1)
            if k - 2 in sc_hs:
                for h in sc_hs.pop(k - 2):
                    h.wait()
            compute_chunk(k)
            sc_hs[k] = fire_scatters(k)
        for hs in sc_hs.values():
            for h in hs:
                h.wait()
        plsc.subcore_barrier()

        def rescale_chunk(k, _):
            px0 = tile_base + k * CHUNK
            row0 = px0 // W
            for ch, dst in enumerate((dxv0, dyv0, dxv1)):
                pltpu.sync_copy(acc_refs[ch].at[pl.ds(px0, CHUNK)], dst)
            pltpu.sync_copy(accW.at[pl.ds(px0, CHUNK)], wbuf)

            def rescale_vec(j, _):
                off = j * L
                orow = off // W
                ocol = off % W
                aw = wbuf[pl.ds(off, L)]
                inv = 1.0 / jnp.where(aw < EPS_W, 1.0, aw)
                imr0[orow, pl.ds(ocol, L)] = dxv0[pl.ds(off, L)] * inv
                img0[orow, pl.ds(ocol, L)] = dyv0[pl.ds(off, L)] * inv
                imb0[orow, pl.ds(ocol, L)] = dxv1[pl.ds(off, L)] * inv
                return 0

            lax.fori_loop(0, CHUNK // L, rescale_vec, 0)
            for ch, src in enumerate((imr0, img0, imb0)):
                pltpu.sync_copy(
                    src, out_hbm.at[img, ch, pl.ds(row0, ROWS), :])
            return 0

        lax.fori_loop(0, NCHUNK, rescale_chunk, 0)
        plsc.subcore_barrier()
        return 0

    lax.fori_loop(0, IMGS_PER_CORE, one_image, 0)


@jax.jit
def kernel(im0, flow):
    flow_flat = jnp.transpose(flow.reshape(B, HW, 2),
                              (0, 2, 1)).reshape(B * 2 * HW)

    mesh = plsc.VectorSubcoreMesh(core_axis_name="c", subcore_axis_name="s",
                                  num_cores=NC, num_subcores=NS)
    f32 = jnp.float32
    i32 = jnp.int32
    warp = pl.kernel(
        _body,
        out_type=jax.ShapeDtypeStruct((B, C, H, W), f32),
        mesh=mesh,
        compiler_params=pltpu.CompilerParams(needs_layout_passes=False,
                                             use_tc_tiling_on_sc=False),
        scratch_types=(
            [pltpu.VMEM_SHARED((HW,), f32)] * 4
            + [pltpu.VMEM((CHUNK,), f32)] * 2
            + [pltpu.VMEM((ROWS, W), f32)] * 3
            + [pltpu.VMEM((CHUNK,), f32)] * 2
            + [pltpu.VMEM((ROWS, W), f32)] * 3
            + [pltpu.VMEM((CHUNK,), i32)] * 8
            + [pltpu.VMEM((4, CHUNK), f32)] * 8
            + [pltpu.VMEM((CHUNK,), f32)] * 2
            + [pltpu.SemaphoreType.DMA] * 3
        ),
    )
    return warp(im0, flow_flat)
